# baseline (speedup 1.0000x reference)
import functools

import jax
import jax.numpy as jnp
from jax import lax
from jax.experimental import pallas as pl
from jax.experimental.pallas import tpu as pltpu
from jax.experimental.pallas import tpu_sc as plsc

_NEG = -1.0e30

_Q = 4096
_E = 128
_DPAD = 100352
_NCHUNK = 784
_QBLK = 256
_DBLK = 1024
_K = 64
_CAP = 128

_NWORKER = 32
_NQ_PER = _Q // _NWORKER
_BATCH = 4
_NBATCH = _NQ_PER // _BATCH


def _sim_body(n_valid, nj, q_ref, d_ref, s_ref, cm_ref):
    j = pl.program_id(0)
    s = lax.dot_general(
        q_ref[...], d_ref[...],
        dimension_numbers=(((1,), (1,)), ((), ())),
        preferred_element_type=jnp.float32,
    )

    def _write(vals):
        s3 = vals.reshape(_QBLK, _DBLK // 128, 128)
        s_ref[...] = s3.reshape(_QBLK, 1, _DBLK // 128, 128)
        cm_ref[...] = jnp.max(s3, axis=2).reshape(1, _QBLK, _DBLK // 128)

    @pl.when(j < nj - 1)
    def _():
        _write(s)

    @pl.when(j == nj - 1)
    def _():
        doc_idx = j * _DBLK + lax.broadcasted_iota(jnp.int32, s.shape, 1)
        _write(jnp.where(doc_idx < n_valid, s, _NEG))


def _similarity(qn, dn, n_valid):
    nj = _DPAD // _DBLK
    ni = _Q // _QBLK
    body = functools.partial(_sim_body, n_valid, nj)
    return pl.pallas_call(
        body,
        grid=(nj, ni),
        in_specs=[
            pl.BlockSpec((_QBLK, _E), lambda j, i: (i, 0)),
            pl.BlockSpec((_DBLK, _E), lambda j, i: (j, 0)),
        ],
        out_specs=[
            pl.BlockSpec((_QBLK, 1, _DBLK // 128, 128),
                         lambda j, i: (i, j, 0, 0)),
            pl.BlockSpec((1, _QBLK, _DBLK // 128), lambda j, i: (j, i, 0)),
        ],
        out_shape=[
            jax.ShapeDtypeStruct((_Q, nj, _DBLK // 128, 128), jnp.float32),
            jax.ShapeDtypeStruct((nj, _Q, _DBLK // 128), jnp.float32),
        ],
    )(qn, dn)


def _select_body(cm_ref, rowid_ref, thr_ref):
    i = pl.program_id(0)
    buf = cm_ref[...]
    lanes = lax.broadcasted_iota(jnp.int32, buf.shape, 1)
    oh = lax.broadcasted_iota(jnp.int32, (_QBLK, _K), 1)
    acc_c = jnp.zeros((_QBLK, _K), jnp.int32)
    m = None
    for it in range(_K):
        m = jnp.max(buf, axis=1, keepdims=True)
        sel = buf == m
        c = jnp.min(jnp.where(sel, lanes, _NCHUNK), axis=1, keepdims=True)
        buf = jnp.where(lanes == c, _NEG, buf)
        acc_c = acc_c + jnp.where(oh == it, c, 0)
    qv = i * _QBLK + lax.broadcasted_iota(jnp.int32, (_QBLK, 1), 0)
    rowid_ref[...] = qv * _NCHUNK + acc_c
    thr_ref[...] = m


def _select_chunks(cm2):
    return pl.pallas_call(
        _select_body,
        grid=(_Q // _QBLK,),
        in_specs=[pl.BlockSpec((_QBLK, _NCHUNK), lambda i: (i, 0))],
        out_specs=[
            pl.BlockSpec((_QBLK, _K), lambda i: (i, 0)),
            pl.BlockSpec((_QBLK, 1), lambda i: (i, 0)),
        ],
        out_shape=[
            jax.ShapeDtypeStruct((_Q, _K), jnp.int32),
            jax.ShapeDtypeStruct((_Q, 1), jnp.float32),
        ],
    )(cm2)


def _sc_gather_filter(s_tab, rowids2, thr, cv_out, ci_out,
                      ids_v, thr_v, rows_v, out_v, out_i, sem0, sem1):
    cid = lax.axis_index("c")
    sid = lax.axis_index("s")
    wid = sid * 2 + cid
    qbase = wid * _NQ_PER
    bbase = wid * _NBATCH

    pltpu.sync_copy(rowids2.at[pl.ds(bbase, _NBATCH)], ids_v)
    pltpu.sync_copy(thr.at[pl.ds(qbase, _NQ_PER)], thr_v.at[pl.ds(0, _NQ_PER)])

    sems = (sem0, sem1)
    nrow = _BATCH * _K

    def _start(k, b):
        pltpu.async_copy(s_tab.at[ids_v.at[k]], rows_v.at[b], sems[b])

    def _wait(k, b):
        pltpu.make_async_copy(s_tab.at[ids_v.at[k]], rows_v.at[b], sems[b]).wait()

    _start(0, 0)
    _start(1, 1)

    iota16 = lax.iota(jnp.int32, 16)
    ones16 = jnp.ones((16,), jnp.int32)
    negv = jnp.full((16,), _NEG, jnp.float32)
    zerov = jnp.zeros((16,), jnp.int32)

    def _process(k, b):
        _wait(k, b)

        def _query(u, carry):
            for i in range(_CAP // 16):
                out_v[u, pl.ds(i * 16, 16)] = negv
                out_i[u, pl.ds(i * 16, 16)] = zerov
            t = thr_v[pl.ds(k * _BATCH + u, 16)][0]
            qg = qbase + k * _BATCH + u

            def _group(g, cnt):
                rvec = ids_v[k, pl.ds(u * _K + g * 16, 16)]
                bases = (rvec - qg * _NCHUNK) * 128
                for rg in range(16):
                    base = bases[rg]
                    row = u * _K + g * 16 + rg
                    for j in range(8):
                        v = rows_v[b, row, pl.ds(j * 16, 16)]
                        mask = v >= t
                        tot = plsc.all_reduce_population_count(mask)[0]
                        idxv = base + j * 16 + iota16
                        plsc.store_compressed(out_v.at[u, pl.ds(cnt, 16)],
                                              v, mask=mask)
                        plsc.store_compressed(out_i.at[u, pl.ds(cnt, 16)],
                                              idxv, mask=mask)
                        cnt = jnp.minimum(cnt + tot, _CAP - 16)
                return cnt

            lax.fori_loop(0, _K // 16, _group, jnp.int32(0))
            return carry

        lax.fori_loop(0, _BATCH, _query, jnp.int32(0))
        pltpu.sync_copy(out_v, cv_out.at[pl.ds(qbase + k * _BATCH, _BATCH)])
        pltpu.sync_copy(out_i, ci_out.at[pl.ds(qbase + k * _BATCH, _BATCH)])

        @pl.when(k + 2 < _NBATCH)
        def _():
            _start(k + 2, b)

    def _pair(h, carry):
        _process(2 * h, 0)
        _process(2 * h + 1, 1)
        return carry

    lax.fori_loop(0, _NBATCH // 2, _pair, jnp.int32(0))


def _sc_candidates(s_tab, rowids2, thr):
    mesh = plsc.VectorSubcoreMesh(core_axis_name="c", subcore_axis_name="s")
    f = functools.partial(
        pl.kernel,
        mesh=mesh,
        compiler_params=pltpu.CompilerParams(
            use_tc_tiling_on_sc=False, needs_layout_passes=False),
        out_type=[
            jax.ShapeDtypeStruct((_Q, _CAP), jnp.float32),
            jax.ShapeDtypeStruct((_Q, _CAP), jnp.int32),
        ],
        scratch_types=[
            pltpu.VMEM((_NBATCH, _BATCH * _K), jnp.int32),
            pltpu.VMEM((_NQ_PER + 16, ), jnp.float32),
            pltpu.VMEM((2, _BATCH * _K, 128), jnp.float32),
            pltpu.VMEM((_BATCH, _CAP), jnp.float32),
            pltpu.VMEM((_BATCH, _CAP), jnp.int32),
            pltpu.SemaphoreType.DMA,
            pltpu.SemaphoreType.DMA,
        ],
    )(_sc_gather_filter)
    return f(s_tab, rowids2, thr)


def _final_body(cv_ref, ci_ref, val_ref, idx_ref):
    buf = cv_ref[...]
    ibuf = ci_ref[...]
    lanes = lax.broadcasted_iota(jnp.int32, buf.shape, 1)
    oh = lax.broadcasted_iota(jnp.int32, (_QBLK, _K), 1)
    vacc = jnp.zeros((_QBLK, _K), jnp.float32)
    iacc = jnp.zeros((_QBLK, _K), jnp.int32)
    big = jnp.int32(2**30)
    for it in range(_K):
        m = jnp.max(buf, axis=1, keepdims=True)
        sel = buf == m
        didx = jnp.min(jnp.where(sel, ibuf, big), axis=1, keepdims=True)
        p = jnp.min(jnp.where(sel & (ibuf == didx), lanes, big),
                    axis=1, keepdims=True)
        buf = jnp.where(lanes == p, _NEG, buf)
        vacc = vacc + jnp.where(oh == it, m, 0.0)
        iacc = iacc + jnp.where(oh == it, didx, 0)
    val_ref[...] = vacc
    idx_ref[...] = iacc


def _final_topk(cv, ci):
    return pl.pallas_call(
        _final_body,
        grid=(_Q // _QBLK,),
        in_specs=[
            pl.BlockSpec((_QBLK, _CAP), lambda i: (i, 0)),
            pl.BlockSpec((_QBLK, _CAP), lambda i: (i, 0)),
        ],
        out_specs=[
            pl.BlockSpec((_QBLK, _K), lambda i: (i, 0)),
            pl.BlockSpec((_QBLK, _K), lambda i: (i, 0)),
        ],
        out_shape=[
            jax.ShapeDtypeStruct((_Q, _K), jnp.float32),
            jax.ShapeDtypeStruct((_Q, _K), jnp.int32),
        ],
    )(cv, ci)


def kernel(query_embed, doc_embeds, k):
    n = doc_embeds.shape[0]

    def _l2n(x):
        nrm = jnp.linalg.norm(x, ord=2, axis=-1, keepdims=True)
        return x / jnp.maximum(nrm, 1e-12)

    qn = _l2n(query_embed)
    dn = jnp.pad(_l2n(doc_embeds), ((0, _DPAD - n), (0, 0)))

    s3, cm = _similarity(qn, dn, n)
    cm2 = jnp.transpose(cm, (1, 0, 2)).reshape(_Q, _NCHUNK)
    rowids, thr = _select_chunks(cm2)

    s_tab = s3.reshape(_Q * _NCHUNK, 128)
    rowids2 = rowids.reshape(_Q // _BATCH, _BATCH * _K)
    cv, ci = _sc_candidates(s_tab, rowids2, thr.reshape(_Q))

    values, indices = _final_topk(cv, ci)
    return (indices, values)


# speedup vs baseline: 1.1585x; 1.1585x over previous
import functools

import jax
import jax.numpy as jnp
from jax import lax
from jax.experimental import pallas as pl
from jax.experimental.pallas import tpu as pltpu
from jax.experimental.pallas import tpu_sc as plsc

_NEG = -1.0e30

_Q = 4096
_E = 128
_DPAD = 100352
_NCHUNK = 784
_QBLK = 256
_DBLK = 1024
_K = 64
_CAP = 128

_NWORKER = 32
_NQ_PER = _Q // _NWORKER
_BATCH = 4
_NBATCH = _NQ_PER // _BATCH


def _sim_body(n_valid, nj, q_ref, d_ref, s_ref, cm_ref):
    j = pl.program_id(0)
    s = lax.dot_general(
        q_ref[...], d_ref[...],
        dimension_numbers=(((1,), (1,)), ((), ())),
        preferred_element_type=jnp.float32,
    )

    def _write(vals):
        s3 = vals.reshape(_QBLK, _DBLK // 128, 128)
        s_ref[...] = s3
        cm_ref[...] = jnp.max(s3, axis=2).reshape(1, _QBLK, _DBLK // 128)

    @pl.when(j < nj - 1)
    def _():
        _write(s)

    @pl.when(j == nj - 1)
    def _():
        doc_idx = j * _DBLK + lax.broadcasted_iota(jnp.int32, s.shape, 1)
        _write(jnp.where(doc_idx < n_valid, s, _NEG))


def _similarity(qn, dn, n_valid):
    nj = _DPAD // _DBLK
    ni = _Q // _QBLK
    body = functools.partial(_sim_body, n_valid, nj)
    return pl.pallas_call(
        body,
        grid=(nj, ni),
        in_specs=[
            pl.BlockSpec((_QBLK, _E), lambda j, i: (i, 0)),
            pl.BlockSpec((_DBLK, _E), lambda j, i: (j, 0)),
        ],
        out_specs=[
            pl.BlockSpec((_QBLK, _DBLK // 128, 128), lambda j, i: (i, j, 0)),
            pl.BlockSpec((1, _QBLK, _DBLK // 128), lambda j, i: (j, i, 0)),
        ],
        out_shape=[
            jax.ShapeDtypeStruct((_Q, _NCHUNK, 128), jnp.float32),
            jax.ShapeDtypeStruct((nj, _Q, _DBLK // 128), jnp.float32),
        ],
    )(qn, dn)


def _select_body(cm_ref, rowid_ref, thr_ref):
    i = pl.program_id(0)
    buf = cm_ref[...]
    lanes = lax.broadcasted_iota(jnp.int32, buf.shape, 1)
    oh = lax.broadcasted_iota(jnp.int32, (_QBLK, _K), 1)
    acc_c = jnp.zeros((_QBLK, _K), jnp.int32)
    m = None
    for it in range(_K):
        m = jnp.max(buf, axis=1, keepdims=True)
        sel = buf == m
        c = jnp.min(jnp.where(sel, lanes, _NCHUNK), axis=1, keepdims=True)
        buf = jnp.where(lanes == c, _NEG, buf)
        acc_c = acc_c + jnp.where(oh == it, c, 0)
    qv = i * _QBLK + lax.broadcasted_iota(jnp.int32, (_QBLK, 1), 0)
    rowid_ref[...] = qv * _NCHUNK + acc_c
    thr_ref[...] = m


def _select_chunks(cm2):
    return pl.pallas_call(
        _select_body,
        grid=(_Q // _QBLK,),
        in_specs=[pl.BlockSpec((_QBLK, _NCHUNK), lambda i: (i, 0))],
        out_specs=[
            pl.BlockSpec((_QBLK, _K), lambda i: (i, 0)),
            pl.BlockSpec((_QBLK, 1), lambda i: (i, 0)),
        ],
        out_shape=[
            jax.ShapeDtypeStruct((_Q, _K), jnp.int32),
            jax.ShapeDtypeStruct((_Q, 1), jnp.float32),
        ],
    )(cm2)


def _sc_gather_filter(s_tab, rowids2, thr, cv_out, ci_out,
                      ids_v, thr_v, rows_v, out_v, out_i, sem0, sem1):
    cid = lax.axis_index("c")
    sid = lax.axis_index("s")
    wid = sid * 2 + cid
    qbase = wid * _NQ_PER
    bbase = wid * _NBATCH

    pltpu.sync_copy(rowids2.at[pl.ds(bbase, _NBATCH)], ids_v)
    pltpu.sync_copy(thr.at[pl.ds(qbase, _NQ_PER)], thr_v.at[pl.ds(0, _NQ_PER)])

    sems = (sem0, sem1)
    nrow = _BATCH * _K

    def _start(k, b):
        pltpu.async_copy(s_tab.at[ids_v.at[k]], rows_v.at[b], sems[b])

    def _wait(k, b):
        pltpu.make_async_copy(s_tab.at[ids_v.at[k]], rows_v.at[b], sems[b]).wait()

    _start(0, 0)
    _start(1, 1)

    iota16 = lax.iota(jnp.int32, 16)
    ones16 = jnp.ones((16,), jnp.int32)
    negv = jnp.full((16,), _NEG, jnp.float32)
    zerov = jnp.zeros((16,), jnp.int32)

    def _process(k, b):
        _wait(k, b)

        def _query(u, carry):
            for i in range(_CAP // 16):
                out_v[u, pl.ds(i * 16, 16)] = negv
                out_i[u, pl.ds(i * 16, 16)] = zerov
            t = thr_v[pl.ds(k * _BATCH + u, 16)][0]
            qg = qbase + k * _BATCH + u

            def _group(g, cnt):
                rvec = ids_v[k, pl.ds(u * _K + g * 16, 16)]
                bases = (rvec - qg * _NCHUNK) * 128
                for rg in range(16):
                    base = bases[rg]
                    row = u * _K + g * 16 + rg
                    for j in range(8):
                        v = rows_v[b, row, pl.ds(j * 16, 16)]
                        mask = v >= t
                        tot = plsc.all_reduce_population_count(mask)[0]
                        idxv = base + j * 16 + iota16
                        plsc.store_compressed(out_v.at[u, pl.ds(cnt, 16)],
                                              v, mask=mask)
                        plsc.store_compressed(out_i.at[u, pl.ds(cnt, 16)],
                                              idxv, mask=mask)
                        cnt = jnp.minimum(cnt + tot, _CAP - 16)
                return cnt

            lax.fori_loop(0, _K // 16, _group, jnp.int32(0))
            return carry

        lax.fori_loop(0, _BATCH, _query, jnp.int32(0))
        pltpu.sync_copy(out_v, cv_out.at[pl.ds(qbase + k * _BATCH, _BATCH)])
        pltpu.sync_copy(out_i, ci_out.at[pl.ds(qbase + k * _BATCH, _BATCH)])

        @pl.when(k + 2 < _NBATCH)
        def _():
            _start(k + 2, b)

    def _pair(h, carry):
        _process(2 * h, 0)
        _process(2 * h + 1, 1)
        return carry

    lax.fori_loop(0, _NBATCH // 2, _pair, jnp.int32(0))


def _sc_candidates(s_tab, rowids2, thr):
    mesh = plsc.VectorSubcoreMesh(core_axis_name="c", subcore_axis_name="s")
    f = functools.partial(
        pl.kernel,
        mesh=mesh,
        compiler_params=pltpu.CompilerParams(
            use_tc_tiling_on_sc=False, needs_layout_passes=False),
        out_type=[
            jax.ShapeDtypeStruct((_Q, _CAP), jnp.float32),
            jax.ShapeDtypeStruct((_Q, _CAP), jnp.int32),
        ],
        scratch_types=[
            pltpu.VMEM((_NBATCH, _BATCH * _K), jnp.int32),
            pltpu.VMEM((_NQ_PER + 16, ), jnp.float32),
            pltpu.VMEM((2, _BATCH * _K, 128), jnp.float32),
            pltpu.VMEM((_BATCH, _CAP), jnp.float32),
            pltpu.VMEM((_BATCH, _CAP), jnp.int32),
            pltpu.SemaphoreType.DMA,
            pltpu.SemaphoreType.DMA,
        ],
    )(_sc_gather_filter)
    return f(s_tab, rowids2, thr)


def _final_body(cv_ref, ci_ref, val_ref, idx_ref):
    buf = cv_ref[...]
    ibuf = ci_ref[...]
    lanes = lax.broadcasted_iota(jnp.int32, buf.shape, 1)
    oh = lax.broadcasted_iota(jnp.int32, (_QBLK, _K), 1)
    vacc = jnp.zeros((_QBLK, _K), jnp.float32)
    iacc = jnp.zeros((_QBLK, _K), jnp.int32)
    big = jnp.int32(2**30)
    for it in range(_K):
        m = jnp.max(buf, axis=1, keepdims=True)
        sel = buf == m
        didx = jnp.min(jnp.where(sel, ibuf, big), axis=1, keepdims=True)
        p = jnp.min(jnp.where(sel & (ibuf == didx), lanes, big),
                    axis=1, keepdims=True)
        buf = jnp.where(lanes == p, _NEG, buf)
        vacc = vacc + jnp.where(oh == it, m, 0.0)
        iacc = iacc + jnp.where(oh == it, didx, 0)
    val_ref[...] = vacc
    idx_ref[...] = iacc


def _final_topk(cv, ci):
    return pl.pallas_call(
        _final_body,
        grid=(_Q // _QBLK,),
        in_specs=[
            pl.BlockSpec((_QBLK, _CAP), lambda i: (i, 0)),
            pl.BlockSpec((_QBLK, _CAP), lambda i: (i, 0)),
        ],
        out_specs=[
            pl.BlockSpec((_QBLK, _K), lambda i: (i, 0)),
            pl.BlockSpec((_QBLK, _K), lambda i: (i, 0)),
        ],
        out_shape=[
            jax.ShapeDtypeStruct((_Q, _K), jnp.float32),
            jax.ShapeDtypeStruct((_Q, _K), jnp.int32),
        ],
    )(cv, ci)


def kernel(query_embed, doc_embeds, k):
    n = doc_embeds.shape[0]

    def _l2n(x):
        nrm = jnp.linalg.norm(x, ord=2, axis=-1, keepdims=True)
        return x / jnp.maximum(nrm, 1e-12)

    qn = _l2n(query_embed)
    dn = jnp.pad(_l2n(doc_embeds), ((0, _DPAD - n), (0, 0)))

    s3, cm = _similarity(qn, dn, n)
    cm2 = jnp.transpose(cm, (1, 0, 2)).reshape(_Q, _NCHUNK)
    rowids, thr = _select_chunks(cm2)

    s_tab = s3.reshape(_Q * _NCHUNK, 128)
    rowids2 = rowids.reshape(_Q // _BATCH, _BATCH * _K)
    cv, ci = _sc_candidates(s_tab, rowids2, thr.reshape(_Q))

    values, indices = _final_topk(cv, ci)
    return (indices, values)


# speedup vs baseline: 1.2268x; 1.0590x over previous
import functools

import jax
import jax.numpy as jnp
from jax import lax
from jax.experimental import pallas as pl
from jax.experimental.pallas import tpu as pltpu
from jax.experimental.pallas import tpu_sc as plsc

_NEG = -1.0e30

_Q = 4096
_E = 128
_DPAD = 100352
_NCHUNK = 784
_QBLK = 256
_DBLK = 1024
_K = 64
_CAP = 128

_NWORKER = 32
_NQ_PER = _Q // _NWORKER
_BATCH = 4
_NBATCH = _NQ_PER // _BATCH


def _sim_body(n_valid, nj, q_ref, d_ref, s_ref, cm_ref):
    j = pl.program_id(0)
    s = lax.dot_general(
        q_ref[...], d_ref[...],
        dimension_numbers=(((1,), (1,)), ((), ())),
        preferred_element_type=jnp.float32,
    )

    def _write(vals):
        s3 = vals.reshape(_QBLK, _DBLK // 128, 128)
        s_ref[...] = s3
        cm_ref[...] = jnp.max(s3, axis=2).reshape(1, _QBLK, _DBLK // 128)

    @pl.when(j < nj - 1)
    def _():
        _write(s)

    @pl.when(j == nj - 1)
    def _():
        doc_idx = j * _DBLK + lax.broadcasted_iota(jnp.int32, s.shape, 1)
        _write(jnp.where(doc_idx < n_valid, s, _NEG))


def _similarity(qn, dn, n_valid):
    nj = _DPAD // _DBLK
    ni = _Q // _QBLK
    body = functools.partial(_sim_body, n_valid, nj)
    return pl.pallas_call(
        body,
        grid=(nj, ni),
        in_specs=[
            pl.BlockSpec((_QBLK, _E), lambda j, i: (i, 0)),
            pl.BlockSpec((_DBLK, _E), lambda j, i: (j, 0)),
        ],
        out_specs=[
            pl.BlockSpec((_QBLK, _DBLK // 128, 128), lambda j, i: (i, j, 0)),
            pl.BlockSpec((1, _QBLK, _DBLK // 128), lambda j, i: (j, i, 0)),
        ],
        out_shape=[
            jax.ShapeDtypeStruct((_Q, _NCHUNK, 128), jnp.float32),
            jax.ShapeDtypeStruct((nj, _Q, _DBLK // 128), jnp.float32),
        ],
    )(qn, dn)


def _thresh_body(cm_ref, thr_ref):
    buf = cm_ref[...]
    b = lax.bitcast_convert_type(buf, jnp.uint32)
    hb = jnp.uint32(0x80000000)
    allb = jnp.uint32(0xFFFFFFFF)
    key = jnp.where(b >= hb, b ^ allb, b | hb)
    lo = jnp.zeros((_QBLK, 1), jnp.uint32)
    hi = jnp.full((_QBLK, 1), allb, jnp.uint32)
    for _ in range(32):
        mid = lo + ((hi - lo) >> jnp.uint32(1))
        cnt = jnp.sum((key >= mid).astype(jnp.int32), axis=1, keepdims=True)
        ge = cnt >= _K
        lo = jnp.where(ge, mid, lo)
        hi = jnp.where(ge, hi, mid)
    tb = jnp.where(lo >= hb, lo & (allb ^ hb), lo ^ allb)
    thr_ref[...] = lax.bitcast_convert_type(tb, jnp.float32)


def _thresholds(cm2):
    return pl.pallas_call(
        _thresh_body,
        grid=(_Q // _QBLK,),
        in_specs=[pl.BlockSpec((_QBLK, _NCHUNK), lambda i: (i, 0))],
        out_specs=pl.BlockSpec((_QBLK, 1), lambda i: (i, 0)),
        out_shape=jax.ShapeDtypeStruct((_Q, 1), jnp.float32),
    )(cm2)


_IDCAP = 80


def _sc_gather_filter(s_tab, cm2, thr, cv_out, ci_out,
                      cm_v, ids_v, thr_v, rows_v, out_v, out_i,
                      gsem0, gsem1, csem0, csem1):
    cid = lax.axis_index("c")
    sid = lax.axis_index("s")
    wid = sid * 2 + cid
    qbase = wid * _NQ_PER

    pltpu.sync_copy(thr.at[pl.ds(qbase, _NQ_PER)], thr_v.at[pl.ds(0, _NQ_PER)])

    gsems = (gsem0, gsem1)
    csems = (csem0, csem1)

    iota16 = lax.iota(jnp.int32, 16)
    negv = jnp.full((16,), _NEG, jnp.float32)
    zerov = jnp.zeros((16,), jnp.int32)

    def _cm_start(k, b):
        pltpu.async_copy(cm2.at[pl.ds(qbase + k * _BATCH, _BATCH)],
                         cm_v.at[b], csems[b])

    def _cm_wait(k, b):
        pltpu.make_async_copy(cm2.at[pl.ds(qbase + k * _BATCH, _BATCH)],
                              cm_v.at[b], csems[b]).wait()

    def _gather_start(k, b):
        for u in range(_BATCH):
            pltpu.async_copy(s_tab.at[ids_v.at[b, u, pl.ds(0, _K)]],
                             rows_v.at[b, pl.ds(u * _K, _K)], gsems[b])

    def _gather_wait(k, b):
        for u in range(_BATCH):
            pltpu.make_async_copy(s_tab.at[ids_v.at[b, u, pl.ds(0, _K)]],
                                  rows_v.at[b, pl.ds(u * _K, _K)],
                                  gsems[b]).wait()

    def _make_ids(k, b):
        def _mq(u, carry):
            t = thr_v[pl.ds(k * _BATCH + u, 16)][0]
            qrow = (qbase + k * _BATCH + u) * _NCHUNK
            for i in range(_IDCAP // 16):
                ids_v[b, u, pl.ds(i * 16, 16)] = qrow + jnp.zeros((16,),
                                                                  jnp.int32)

            def _g7(g, cnt):
                for g2 in range(7):
                    gg = g * 7 + g2
                    v = cm_v[b, u, pl.ds(gg * 16, 16)]
                    mask = v >= t
                    tot = plsc.all_reduce_population_count(mask)[0]
                    rowv = qrow + gg * 16 + iota16
                    plsc.store_compressed(ids_v.at[b, u, pl.ds(cnt, 16)],
                                          rowv, mask=mask)
                    cnt = jnp.minimum(cnt + tot, _IDCAP - 16)
                return cnt

            lax.fori_loop(0, 7, _g7, jnp.int32(0))
            return carry

        lax.fori_loop(0, _BATCH, _mq, jnp.int32(0))

    _cm_start(0, 0)
    _cm_start(1, 1)
    _cm_wait(0, 0)
    _make_ids(0, 0)
    _gather_start(0, 0)
    _cm_wait(1, 1)
    _make_ids(1, 1)
    _gather_start(1, 1)
    _cm_start(2, 0)
    _cm_start(3, 1)

    def _process(k, b):
        _gather_wait(k, b)

        def _query(u, carry):
            for i in range(_CAP // 16):
                out_v[u, pl.ds(i * 16, 16)] = negv
                out_i[u, pl.ds(i * 16, 16)] = zerov
            t = thr_v[pl.ds(k * _BATCH + u, 16)][0]
            qg = qbase + k * _BATCH + u

            def _group(g, cnt):
                rvec = ids_v[b, u, pl.ds(g * 16, 16)]
                bases = (rvec - qg * _NCHUNK) * 128
                for rg in range(16):
                    base = bases[rg]
                    row = u * _K + g * 16 + rg
                    for j in range(8):
                        v = rows_v[b, row, pl.ds(j * 16, 16)]
                        mask = v >= t
                        tot = plsc.all_reduce_population_count(mask)[0]
                        idxv = base + j * 16 + iota16
                        plsc.store_compressed(out_v.at[u, pl.ds(cnt, 16)],
                                              v, mask=mask)
                        plsc.store_compressed(out_i.at[u, pl.ds(cnt, 16)],
                                              idxv, mask=mask)
                        cnt = jnp.minimum(cnt + tot, _CAP - 16)
                return cnt

            lax.fori_loop(0, _K // 16, _group, jnp.int32(0))
            return carry

        lax.fori_loop(0, _BATCH, _query, jnp.int32(0))
        pltpu.sync_copy(out_v, cv_out.at[pl.ds(qbase + k * _BATCH, _BATCH)])
        pltpu.sync_copy(out_i, ci_out.at[pl.ds(qbase + k * _BATCH, _BATCH)])

        @pl.when(k + 2 < _NBATCH)
        def _():
            _cm_wait(k + 2, b)
            _make_ids(k + 2, b)
            _gather_start(k + 2, b)

        @pl.when(k + 4 < _NBATCH)
        def _():
            _cm_start(k + 4, b)

    def _pair(h, carry):
        _process(2 * h, 0)
        _process(2 * h + 1, 1)
        return carry

    lax.fori_loop(0, _NBATCH // 2, _pair, jnp.int32(0))


def _sc_candidates(s_tab, cm2, thr):
    mesh = plsc.VectorSubcoreMesh(core_axis_name="c", subcore_axis_name="s")
    f = functools.partial(
        pl.kernel,
        mesh=mesh,
        compiler_params=pltpu.CompilerParams(
            use_tc_tiling_on_sc=False, needs_layout_passes=False),
        out_type=[
            jax.ShapeDtypeStruct((_Q, _CAP), jnp.float32),
            jax.ShapeDtypeStruct((_Q, _CAP), jnp.int32),
        ],
        scratch_types=[
            pltpu.VMEM((2, _BATCH, _NCHUNK), jnp.float32),
            pltpu.VMEM((2, _BATCH, _IDCAP), jnp.int32),
            pltpu.VMEM((_NQ_PER + 16, ), jnp.float32),
            pltpu.VMEM((2, _BATCH * _K, 128), jnp.float32),
            pltpu.VMEM((_BATCH, _CAP), jnp.float32),
            pltpu.VMEM((_BATCH, _CAP), jnp.int32),
            pltpu.SemaphoreType.DMA,
            pltpu.SemaphoreType.DMA,
            pltpu.SemaphoreType.DMA,
            pltpu.SemaphoreType.DMA,
        ],
    )(_sc_gather_filter)
    return f(s_tab, cm2, thr)


def _final_body(cv_ref, ci_ref, val_ref, idx_ref):
    buf = cv_ref[...]
    ibuf = ci_ref[...]
    lanes = lax.broadcasted_iota(jnp.int32, buf.shape, 1)
    oh = lax.broadcasted_iota(jnp.int32, (_QBLK, _K), 1)
    vacc = jnp.zeros((_QBLK, _K), jnp.float32)
    iacc = jnp.zeros((_QBLK, _K), jnp.int32)
    big = jnp.int32(2**30)
    for it in range(_K):
        m = jnp.max(buf, axis=1, keepdims=True)
        sel = buf == m
        didx = jnp.min(jnp.where(sel, ibuf, big), axis=1, keepdims=True)
        p = jnp.min(jnp.where(sel & (ibuf == didx), lanes, big),
                    axis=1, keepdims=True)
        buf = jnp.where(lanes == p, _NEG, buf)
        vacc = vacc + jnp.where(oh == it, m, 0.0)
        iacc = iacc + jnp.where(oh == it, didx, 0)
    val_ref[...] = vacc
    idx_ref[...] = iacc


def _final_topk(cv, ci):
    return pl.pallas_call(
        _final_body,
        grid=(_Q // _QBLK,),
        in_specs=[
            pl.BlockSpec((_QBLK, _CAP), lambda i: (i, 0)),
            pl.BlockSpec((_QBLK, _CAP), lambda i: (i, 0)),
        ],
        out_specs=[
            pl.BlockSpec((_QBLK, _K), lambda i: (i, 0)),
            pl.BlockSpec((_QBLK, _K), lambda i: (i, 0)),
        ],
        out_shape=[
            jax.ShapeDtypeStruct((_Q, _K), jnp.float32),
            jax.ShapeDtypeStruct((_Q, _K), jnp.int32),
        ],
    )(cv, ci)


def kernel(query_embed, doc_embeds, k):
    n = doc_embeds.shape[0]

    def _l2n(x):
        nrm = jnp.linalg.norm(x, ord=2, axis=-1, keepdims=True)
        return x / jnp.maximum(nrm, 1e-12)

    qn = _l2n(query_embed)
    dn = jnp.pad(_l2n(doc_embeds), ((0, _DPAD - n), (0, 0)))

    s3, cm = _similarity(qn, dn, n)
    cm2 = jnp.transpose(cm, (1, 0, 2)).reshape(_Q, _NCHUNK)
    thr = _thresholds(cm2)

    s_tab = s3.reshape(_Q * _NCHUNK, 128)
    cv, ci = _sc_candidates(s_tab, cm2, thr.reshape(_Q))

    values, indices = _final_topk(cv, ci)
    return (indices, values)


# speedup vs baseline: 1.2277x; 1.0007x over previous
import functools

import jax
import jax.numpy as jnp
from jax import lax
from jax.experimental import pallas as pl
from jax.experimental.pallas import tpu as pltpu
from jax.experimental.pallas import tpu_sc as plsc

_NEG = -1.0e30

_Q = 4096
_E = 128
_DPAD = 100352
_NCHUNK = 784
_QBLK = 256
_DBLK = 1024
_K = 64
_CAP = 128

_NWORKER = 32
_NQ_PER = _Q // _NWORKER
_BATCH = 4
_NBATCH = _NQ_PER // _BATCH


def _sim_body(n_valid, nj, q_ref, d_ref, s_ref, cm_ref):
    j = pl.program_id(0)
    s = lax.dot_general(
        q_ref[...], d_ref[...],
        dimension_numbers=(((1,), (1,)), ((), ())),
        preferred_element_type=jnp.float32,
    )

    def _write(vals):
        s3 = vals.reshape(_QBLK, _DBLK // 128, 128)
        s_ref[...] = s3
        cm_ref[...] = jnp.max(s3, axis=2).reshape(1, _QBLK, _DBLK // 128)

    @pl.when(j < nj - 1)
    def _():
        _write(s)

    @pl.when(j == nj - 1)
    def _():
        doc_idx = j * _DBLK + lax.broadcasted_iota(jnp.int32, s.shape, 1)
        _write(jnp.where(doc_idx < n_valid, s, _NEG))


def _similarity(qn, dn, n_valid):
    nj = _DPAD // _DBLK
    ni = _Q // _QBLK
    body = functools.partial(_sim_body, n_valid, nj)
    return pl.pallas_call(
        body,
        grid=(nj, ni),
        in_specs=[
            pl.BlockSpec((_QBLK, _E), lambda j, i: (i, 0)),
            pl.BlockSpec((_DBLK, _E), lambda j, i: (j, 0)),
        ],
        out_specs=[
            pl.BlockSpec((_QBLK, _DBLK // 128, 128), lambda j, i: (i, j, 0)),
            pl.BlockSpec((1, _QBLK, _DBLK // 128), lambda j, i: (j, i, 0)),
        ],
        out_shape=[
            jax.ShapeDtypeStruct((_Q, _NCHUNK, 128), jnp.float32),
            jax.ShapeDtypeStruct((nj, _Q, _DBLK // 128), jnp.float32),
        ],
    )(qn, dn)


def _thresh_body(cm_ref, thr_ref):
    buf = cm_ref[...]
    b = lax.bitcast_convert_type(buf, jnp.int32)
    imin = jnp.int32(-0x80000000)
    key = jnp.where(b >= 0, b, imin - b)
    bound = jnp.int32(0x3F8147AE)
    lo = jnp.full((_QBLK, 1), -bound, jnp.int32)
    hi = jnp.full((_QBLK, 1), bound, jnp.int32)
    for _ in range(31):
        mid = lo + ((hi - lo) >> 1)
        cnt = jnp.sum((key >= mid).astype(jnp.int32), axis=1, keepdims=True)
        ge = cnt >= _K
        lo = jnp.where(ge, mid, lo)
        hi = jnp.where(ge, hi, mid)
    tb = jnp.where(lo >= 0, lo, imin - lo)
    thr_ref[...] = lax.bitcast_convert_type(tb, jnp.float32)


def _thresholds(cm2):
    return pl.pallas_call(
        _thresh_body,
        grid=(_Q // _QBLK,),
        in_specs=[pl.BlockSpec((_QBLK, _NCHUNK), lambda i: (i, 0))],
        out_specs=pl.BlockSpec((_QBLK, 1), lambda i: (i, 0)),
        out_shape=jax.ShapeDtypeStruct((_Q, 1), jnp.float32),
    )(cm2)


_IDCAP = 80


def _sc_gather_filter(s_tab, cm2, thr, cv_out, ci_out,
                      cm_v, ids_v, thr_v, rows_v, out_v, out_i,
                      gsem0, gsem1, csem0, csem1):
    cid = lax.axis_index("c")
    sid = lax.axis_index("s")
    wid = sid * 2 + cid
    qbase = wid * _NQ_PER

    pltpu.sync_copy(thr.at[pl.ds(qbase, _NQ_PER)], thr_v.at[pl.ds(0, _NQ_PER)])

    gsems = (gsem0, gsem1)
    csems = (csem0, csem1)

    iota16 = lax.iota(jnp.int32, 16)
    negv = jnp.full((16,), _NEG, jnp.float32)
    zerov = jnp.zeros((16,), jnp.int32)

    def _cm_start(k, b):
        pltpu.async_copy(cm2.at[pl.ds(qbase + k * _BATCH, _BATCH)],
                         cm_v.at[b], csems[b])

    def _cm_wait(k, b):
        pltpu.make_async_copy(cm2.at[pl.ds(qbase + k * _BATCH, _BATCH)],
                              cm_v.at[b], csems[b]).wait()

    def _gather_start(k, b):
        for u in range(_BATCH):
            pltpu.async_copy(s_tab.at[ids_v.at[b, u, pl.ds(0, _K)]],
                             rows_v.at[b, pl.ds(u * _K, _K)], gsems[b])

    def _gather_wait(k, b):
        for u in range(_BATCH):
            pltpu.make_async_copy(s_tab.at[ids_v.at[b, u, pl.ds(0, _K)]],
                                  rows_v.at[b, pl.ds(u * _K, _K)],
                                  gsems[b]).wait()

    def _make_ids(k, b):
        def _mq(u, carry):
            t = thr_v[pl.ds(k * _BATCH + u, 16)][0]
            qrow = (qbase + k * _BATCH + u) * _NCHUNK
            for i in range(_IDCAP // 16):
                ids_v[b, u, pl.ds(i * 16, 16)] = qrow + jnp.zeros((16,),
                                                                  jnp.int32)

            def _g7(g, cnt):
                for g2 in range(7):
                    gg = g * 7 + g2
                    v = cm_v[b, u, pl.ds(gg * 16, 16)]
                    mask = v >= t
                    tot = plsc.all_reduce_population_count(mask)[0]
                    rowv = qrow + gg * 16 + iota16
                    plsc.store_compressed(ids_v.at[b, u, pl.ds(cnt, 16)],
                                          rowv, mask=mask)
                    cnt = jnp.minimum(cnt + tot, _IDCAP - 16)
                return cnt

            lax.fori_loop(0, 7, _g7, jnp.int32(0))
            return carry

        lax.fori_loop(0, _BATCH, _mq, jnp.int32(0))

    _cm_start(0, 0)
    _cm_start(1, 1)
    _cm_wait(0, 0)
    _make_ids(0, 0)
    _gather_start(0, 0)
    _cm_wait(1, 1)
    _make_ids(1, 1)
    _gather_start(1, 1)
    _cm_start(2, 0)
    _cm_start(3, 1)

    def _process(k, b):
        _gather_wait(k, b)

        def _query(u, carry):
            for i in range(_CAP // 16):
                out_v[u, pl.ds(i * 16, 16)] = negv
                out_i[u, pl.ds(i * 16, 16)] = zerov
            t = thr_v[pl.ds(k * _BATCH + u, 16)][0]
            qg = qbase + k * _BATCH + u

            def _group(g, cnt):
                rvec = ids_v[b, u, pl.ds(g * 16, 16)]
                bases = (rvec - qg * _NCHUNK) * 128
                for rg in range(16):
                    base = bases[rg]
                    row = u * _K + g * 16 + rg
                    for j in range(8):
                        v = rows_v[b, row, pl.ds(j * 16, 16)]
                        mask = v >= t
                        tot = plsc.all_reduce_population_count(mask)[0]
                        idxv = base + j * 16 + iota16
                        plsc.store_compressed(out_v.at[u, pl.ds(cnt, 16)],
                                              v, mask=mask)
                        plsc.store_compressed(out_i.at[u, pl.ds(cnt, 16)],
                                              idxv, mask=mask)
                        cnt = jnp.minimum(cnt + tot, _CAP - 16)
                return cnt

            lax.fori_loop(0, _K // 16, _group, jnp.int32(0))
            return carry

        lax.fori_loop(0, _BATCH, _query, jnp.int32(0))
        pltpu.sync_copy(out_v, cv_out.at[pl.ds(qbase + k * _BATCH, _BATCH)])
        pltpu.sync_copy(out_i, ci_out.at[pl.ds(qbase + k * _BATCH, _BATCH)])

        @pl.when(k + 2 < _NBATCH)
        def _():
            _cm_wait(k + 2, b)
            _make_ids(k + 2, b)
            _gather_start(k + 2, b)

        @pl.when(k + 4 < _NBATCH)
        def _():
            _cm_start(k + 4, b)

    def _pair(h, carry):
        _process(2 * h, 0)
        _process(2 * h + 1, 1)
        return carry

    lax.fori_loop(0, _NBATCH // 2, _pair, jnp.int32(0))


def _sc_candidates(s_tab, cm2, thr):
    mesh = plsc.VectorSubcoreMesh(core_axis_name="c", subcore_axis_name="s")
    f = functools.partial(
        pl.kernel,
        mesh=mesh,
        compiler_params=pltpu.CompilerParams(
            use_tc_tiling_on_sc=False, needs_layout_passes=False),
        out_type=[
            jax.ShapeDtypeStruct((_Q, _CAP), jnp.float32),
            jax.ShapeDtypeStruct((_Q, _CAP), jnp.int32),
        ],
        scratch_types=[
            pltpu.VMEM((2, _BATCH, _NCHUNK), jnp.float32),
            pltpu.VMEM((2, _BATCH, _IDCAP), jnp.int32),
            pltpu.VMEM((_NQ_PER + 16, ), jnp.float32),
            pltpu.VMEM((2, _BATCH * _K, 128), jnp.float32),
            pltpu.VMEM((_BATCH, _CAP), jnp.float32),
            pltpu.VMEM((_BATCH, _CAP), jnp.int32),
            pltpu.SemaphoreType.DMA,
            pltpu.SemaphoreType.DMA,
            pltpu.SemaphoreType.DMA,
            pltpu.SemaphoreType.DMA,
        ],
    )(_sc_gather_filter)
    return f(s_tab, cm2, thr)


def _final_body(cv_ref, ci_ref, val_ref, idx_ref):
    buf = cv_ref[...]
    ibuf = ci_ref[...]
    lanes = lax.broadcasted_iota(jnp.int32, buf.shape, 1)
    oh = lax.broadcasted_iota(jnp.int32, (_QBLK, _K), 1)
    vacc = jnp.zeros((_QBLK, _K), jnp.float32)
    iacc = jnp.zeros((_QBLK, _K), jnp.int32)
    big = jnp.int32(2**30)
    for it in range(_K):
        m = jnp.max(buf, axis=1, keepdims=True)
        sel = buf == m
        didx = jnp.min(jnp.where(sel, ibuf, big), axis=1, keepdims=True)
        p = jnp.min(jnp.where(sel & (ibuf == didx), lanes, big),
                    axis=1, keepdims=True)
        buf = jnp.where(lanes == p, _NEG, buf)
        vacc = vacc + jnp.where(oh == it, m, 0.0)
        iacc = iacc + jnp.where(oh == it, didx, 0)
    val_ref[...] = vacc
    idx_ref[...] = iacc


def _final_topk(cv, ci):
    return pl.pallas_call(
        _final_body,
        grid=(_Q // _QBLK,),
        in_specs=[
            pl.BlockSpec((_QBLK, _CAP), lambda i: (i, 0)),
            pl.BlockSpec((_QBLK, _CAP), lambda i: (i, 0)),
        ],
        out_specs=[
            pl.BlockSpec((_QBLK, _K), lambda i: (i, 0)),
            pl.BlockSpec((_QBLK, _K), lambda i: (i, 0)),
        ],
        out_shape=[
            jax.ShapeDtypeStruct((_Q, _K), jnp.float32),
            jax.ShapeDtypeStruct((_Q, _K), jnp.int32),
        ],
    )(cv, ci)


def kernel(query_embed, doc_embeds, k):
    n = doc_embeds.shape[0]

    def _l2n(x):
        nrm = jnp.linalg.norm(x, ord=2, axis=-1, keepdims=True)
        return x / jnp.maximum(nrm, 1e-12)

    qn = _l2n(query_embed)
    dn = jnp.pad(_l2n(doc_embeds), ((0, _DPAD - n), (0, 0)))

    s3, cm = _similarity(qn, dn, n)
    cm2 = jnp.transpose(cm, (1, 0, 2)).reshape(_Q, _NCHUNK)
    thr = _thresholds(cm2)

    s_tab = s3.reshape(_Q * _NCHUNK, 128)
    cv, ci = _sc_candidates(s_tab, cm2, thr.reshape(_Q))

    values, indices = _final_topk(cv, ci)
    return (indices, values)


# speedup vs baseline: 1.2833x; 1.0454x over previous
import functools

import jax
import jax.numpy as jnp
from jax import lax
from jax.experimental import pallas as pl
from jax.experimental.pallas import tpu as pltpu
from jax.experimental.pallas import tpu_sc as plsc

_NEG = -1.0e30

_Q = 4096
_E = 128
_DPAD = 100352
_NCHUNK = 784
_QBLK = 256
_DBLK = 1024
_K = 64
_CAP = 128

_NWORKER = 32
_NQ_PER = _Q // _NWORKER
_BATCH = 4
_NBATCH = _NQ_PER // _BATCH


def _sim_body(n_valid, nj, q_ref, d_ref, s_ref, cm_ref):
    j = pl.program_id(0)
    s = lax.dot_general(
        q_ref[...], d_ref[...],
        dimension_numbers=(((1,), (1,)), ((), ())),
        preferred_element_type=jnp.float32,
    )

    def _write(vals):
        s3 = vals.reshape(_QBLK, _DBLK // 128, 128)
        s_ref[...] = jnp.swapaxes(s3, 0, 1)
        cm_ref[...] = jnp.max(s3, axis=2).reshape(1, _QBLK, _DBLK // 128)

    @pl.when(j < nj - 1)
    def _():
        _write(s)

    @pl.when(j == nj - 1)
    def _():
        doc_idx = j * _DBLK + lax.broadcasted_iota(jnp.int32, s.shape, 1)
        _write(jnp.where(doc_idx < n_valid, s, _NEG))


def _similarity(qn, dn, n_valid):
    nj = _DPAD // _DBLK
    ni = _Q // _QBLK
    body = functools.partial(_sim_body, n_valid, nj)
    return pl.pallas_call(
        body,
        grid=(nj, ni),
        in_specs=[
            pl.BlockSpec((_QBLK, _E), lambda j, i: (i, 0)),
            pl.BlockSpec((_DBLK, _E), lambda j, i: (j, 0)),
        ],
        out_specs=[
            pl.BlockSpec((_DBLK // 128, _QBLK, 128), lambda j, i: (j, i, 0)),
            pl.BlockSpec((1, _QBLK, _DBLK // 128), lambda j, i: (j, i, 0)),
        ],
        out_shape=[
            jax.ShapeDtypeStruct((_NCHUNK, _Q, 128), jnp.float32),
            jax.ShapeDtypeStruct((nj, _Q, _DBLK // 128), jnp.float32),
        ],
    )(qn, dn)


def _thresh_body(cm_ref, thr_ref):
    buf = cm_ref[...]
    b = lax.bitcast_convert_type(buf, jnp.int32)
    imin = jnp.int32(-0x80000000)
    key = jnp.where(b >= 0, b, imin - b)
    bound = jnp.int32(0x3F8147AE)
    lo = jnp.full((_QBLK, 1), -bound, jnp.int32)
    hi = jnp.full((_QBLK, 1), bound, jnp.int32)
    for _ in range(31):
        mid = lo + ((hi - lo) >> 1)
        cnt = jnp.sum((key >= mid).astype(jnp.int32), axis=1, keepdims=True)
        ge = cnt >= _K
        lo = jnp.where(ge, mid, lo)
        hi = jnp.where(ge, hi, mid)
    tb = jnp.where(lo >= 0, lo, imin - lo)
    thr_ref[...] = lax.bitcast_convert_type(tb, jnp.float32)


def _thresholds(cm2):
    return pl.pallas_call(
        _thresh_body,
        grid=(_Q // _QBLK,),
        in_specs=[pl.BlockSpec((_QBLK, _NCHUNK), lambda i: (i, 0))],
        out_specs=pl.BlockSpec((_QBLK, 1), lambda i: (i, 0)),
        out_shape=jax.ShapeDtypeStruct((_Q, 1), jnp.float32),
    )(cm2)


_IDCAP = 80


def _sc_gather_filter(s_tab, cm2, thr, cv_out, ci_out,
                      cm_v, ids_v, thr_v, rows_v, out_v, out_i,
                      gsem0, gsem1, csem0, csem1):
    cid = lax.axis_index("c")
    sid = lax.axis_index("s")
    wid = sid * 2 + cid
    qbase = wid * _NQ_PER

    pltpu.sync_copy(thr.at[pl.ds(qbase, _NQ_PER)], thr_v.at[pl.ds(0, _NQ_PER)])

    gsems = (gsem0, gsem1)
    csems = (csem0, csem1)

    iota16 = lax.iota(jnp.int32, 16)
    negv = jnp.full((16,), _NEG, jnp.float32)
    zerov = jnp.zeros((16,), jnp.int32)

    def _cm_start(k, b):
        pltpu.async_copy(cm2.at[pl.ds(qbase + k * _BATCH, _BATCH)],
                         cm_v.at[b], csems[b])

    def _cm_wait(k, b):
        pltpu.make_async_copy(cm2.at[pl.ds(qbase + k * _BATCH, _BATCH)],
                              cm_v.at[b], csems[b]).wait()

    def _gather_start(k, b):
        for u in range(_BATCH):
            pltpu.async_copy(s_tab.at[ids_v.at[b, u, pl.ds(0, _K)]],
                             rows_v.at[b, pl.ds(u * _K, _K)], gsems[b])

    def _gather_wait(k, b):
        for u in range(_BATCH):
            pltpu.make_async_copy(s_tab.at[ids_v.at[b, u, pl.ds(0, _K)]],
                                  rows_v.at[b, pl.ds(u * _K, _K)],
                                  gsems[b]).wait()

    def _make_ids(k, b):
        def _mq(u, carry):
            t = thr_v[pl.ds(k * _BATCH + u, 16)][0]
            qg = qbase + k * _BATCH + u
            for i in range(_IDCAP // 16):
                ids_v[b, u, pl.ds(i * 16, 16)] = qg + jnp.zeros((16,),
                                                                jnp.int32)

            def _g7(g, cnt):
                for g2 in range(7):
                    gg = g * 7 + g2
                    v = cm_v[b, u, pl.ds(gg * 16, 16)]
                    mask = v >= t
                    tot = plsc.all_reduce_population_count(mask)[0]
                    rowv = (gg * 16 + iota16) * _Q + qg
                    plsc.store_compressed(ids_v.at[b, u, pl.ds(cnt, 16)],
                                          rowv, mask=mask)
                    cnt = jnp.minimum(cnt + tot, _IDCAP - 16)
                return cnt

            lax.fori_loop(0, 7, _g7, jnp.int32(0))
            return carry

        lax.fori_loop(0, _BATCH, _mq, jnp.int32(0))

    _cm_start(0, 0)
    _cm_start(1, 1)
    _cm_wait(0, 0)
    _make_ids(0, 0)
    _gather_start(0, 0)
    _cm_wait(1, 1)
    _make_ids(1, 1)
    _gather_start(1, 1)
    _cm_start(2, 0)
    _cm_start(3, 1)

    def _process(k, b):
        _gather_wait(k, b)

        def _query(u, carry):
            for i in range(_CAP // 16):
                out_v[u, pl.ds(i * 16, 16)] = negv
                out_i[u, pl.ds(i * 16, 16)] = zerov
            t = thr_v[pl.ds(k * _BATCH + u, 16)][0]
            qg = qbase + k * _BATCH + u

            def _group(g, cnt):
                rvec = ids_v[b, u, pl.ds(g * 16, 16)]
                bases = (rvec - qg) >> 5
                for rg in range(16):
                    base = bases[rg]
                    row = u * _K + g * 16 + rg
                    for j in range(8):
                        v = rows_v[b, row, pl.ds(j * 16, 16)]
                        mask = v >= t
                        tot = plsc.all_reduce_population_count(mask)[0]
                        idxv = base + j * 16 + iota16
                        plsc.store_compressed(out_v.at[u, pl.ds(cnt, 16)],
                                              v, mask=mask)
                        plsc.store_compressed(out_i.at[u, pl.ds(cnt, 16)],
                                              idxv, mask=mask)
                        cnt = jnp.minimum(cnt + tot, _CAP - 16)
                return cnt

            lax.fori_loop(0, _K // 16, _group, jnp.int32(0))
            return carry

        lax.fori_loop(0, _BATCH, _query, jnp.int32(0))
        pltpu.sync_copy(out_v, cv_out.at[pl.ds(qbase + k * _BATCH, _BATCH)])
        pltpu.sync_copy(out_i, ci_out.at[pl.ds(qbase + k * _BATCH, _BATCH)])

        @pl.when(k + 2 < _NBATCH)
        def _():
            _cm_wait(k + 2, b)
            _make_ids(k + 2, b)
            _gather_start(k + 2, b)

        @pl.when(k + 4 < _NBATCH)
        def _():
            _cm_start(k + 4, b)

    def _pair(h, carry):
        _process(2 * h, 0)
        _process(2 * h + 1, 1)
        return carry

    lax.fori_loop(0, _NBATCH // 2, _pair, jnp.int32(0))


def _sc_candidates(s_tab, cm2, thr):
    mesh = plsc.VectorSubcoreMesh(core_axis_name="c", subcore_axis_name="s")
    f = functools.partial(
        pl.kernel,
        mesh=mesh,
        compiler_params=pltpu.CompilerParams(
            use_tc_tiling_on_sc=False, needs_layout_passes=False),
        out_type=[
            jax.ShapeDtypeStruct((_Q, _CAP), jnp.float32),
            jax.ShapeDtypeStruct((_Q, _CAP), jnp.int32),
        ],
        scratch_types=[
            pltpu.VMEM((2, _BATCH, _NCHUNK), jnp.float32),
            pltpu.VMEM((2, _BATCH, _IDCAP), jnp.int32),
            pltpu.VMEM((_NQ_PER + 16, ), jnp.float32),
            pltpu.VMEM((2, _BATCH * _K, 128), jnp.float32),
            pltpu.VMEM((_BATCH, _CAP), jnp.float32),
            pltpu.VMEM((_BATCH, _CAP), jnp.int32),
            pltpu.SemaphoreType.DMA,
            pltpu.SemaphoreType.DMA,
            pltpu.SemaphoreType.DMA,
            pltpu.SemaphoreType.DMA,
        ],
    )(_sc_gather_filter)
    return f(s_tab, cm2, thr)


def _final_body(cv_ref, ci_ref, val_ref, idx_ref):
    buf = cv_ref[...]
    ibuf = ci_ref[...]
    lanes = lax.broadcasted_iota(jnp.int32, buf.shape, 1)
    oh = lax.broadcasted_iota(jnp.int32, (_QBLK, _K), 1)
    vacc = jnp.zeros((_QBLK, _K), jnp.float32)
    iacc = jnp.zeros((_QBLK, _K), jnp.int32)
    big = jnp.int32(2**30)
    for it in range(_K):
        m = jnp.max(buf, axis=1, keepdims=True)
        sel = buf == m
        didx = jnp.min(jnp.where(sel, ibuf, big), axis=1, keepdims=True)
        buf = jnp.where(sel & (ibuf == didx), _NEG, buf)
        vacc = vacc + jnp.where(oh == it, m, 0.0)
        iacc = iacc + jnp.where(oh == it, didx, 0)
    val_ref[...] = vacc
    idx_ref[...] = iacc


def _final_topk(cv, ci):
    return pl.pallas_call(
        _final_body,
        grid=(_Q // _QBLK,),
        in_specs=[
            pl.BlockSpec((_QBLK, _CAP), lambda i: (i, 0)),
            pl.BlockSpec((_QBLK, _CAP), lambda i: (i, 0)),
        ],
        out_specs=[
            pl.BlockSpec((_QBLK, _K), lambda i: (i, 0)),
            pl.BlockSpec((_QBLK, _K), lambda i: (i, 0)),
        ],
        out_shape=[
            jax.ShapeDtypeStruct((_Q, _K), jnp.float32),
            jax.ShapeDtypeStruct((_Q, _K), jnp.int32),
        ],
    )(cv, ci)


def kernel(query_embed, doc_embeds, k):
    n = doc_embeds.shape[0]

    def _l2n(x):
        nrm = jnp.linalg.norm(x, ord=2, axis=-1, keepdims=True)
        return x / jnp.maximum(nrm, 1e-12)

    qn = _l2n(query_embed)
    dn = jnp.pad(_l2n(doc_embeds), ((0, _DPAD - n), (0, 0)))

    s3, cm = _similarity(qn, dn, n)
    cm2 = jnp.transpose(cm, (1, 0, 2)).reshape(_Q, _NCHUNK)
    thr = _thresholds(cm2)

    s_tab = s3.reshape(_NCHUNK * _Q, 128)
    cv, ci = _sc_candidates(s_tab, cm2, thr.reshape(_Q))

    values, indices = _final_topk(cv, ci)
    return (indices, values)


# speedup vs baseline: 1.5060x; 1.1735x over previous
import functools

import jax
import jax.numpy as jnp
from jax import lax
from jax.experimental import pallas as pl
from jax.experimental.pallas import tpu as pltpu
from jax.experimental.pallas import tpu_sc as plsc

_NEG = -1.0e30

_Q = 4096
_E = 128
_DPAD = 100352
_NCHUNK = 784
_QBLK = 256
_DBLK = 2048
_K = 64
_CAP = 128

_NWORKER = 32
_NQ_PER = _Q // _NWORKER
_BATCH = 4
_NBATCH = _NQ_PER // _BATCH


def _sim_body(n_valid, nj, q_ref, d_ref, s_ref, cm_ref):
    j = pl.program_id(0)
    s = lax.dot_general(
        q_ref[...], d_ref[...],
        dimension_numbers=(((1,), (1,)), ((), ())),
        preferred_element_type=jnp.float32,
    )

    def _write(vals):
        s3 = vals.reshape(_QBLK, _DBLK // 128, 128)
        s_ref[...] = jnp.swapaxes(s3, 0, 1)
        cm_ref[...] = jnp.max(s3, axis=2).reshape(1, _QBLK, _DBLK // 128)

    @pl.when(j < nj - 1)
    def _():
        _write(s)

    @pl.when(j == nj - 1)
    def _():
        doc_idx = j * _DBLK + lax.broadcasted_iota(jnp.int32, s.shape, 1)
        _write(jnp.where(doc_idx < n_valid, s, _NEG))


def _similarity(qn, dn, n_valid):
    nj = _DPAD // _DBLK
    ni = _Q // _QBLK
    body = functools.partial(_sim_body, n_valid, nj)
    return pl.pallas_call(
        body,
        grid=(nj, ni),
        in_specs=[
            pl.BlockSpec((_QBLK, _E), lambda j, i: (i, 0)),
            pl.BlockSpec((_DBLK, _E), lambda j, i: (j, 0)),
        ],
        out_specs=[
            pl.BlockSpec((_DBLK // 128, _QBLK, 128), lambda j, i: (j, i, 0)),
            pl.BlockSpec((1, _QBLK, _DBLK // 128), lambda j, i: (j, i, 0)),
        ],
        out_shape=[
            jax.ShapeDtypeStruct((_NCHUNK, _Q, 128), jnp.float32),
            jax.ShapeDtypeStruct((nj, _Q, _DBLK // 128), jnp.float32),
        ],
    )(qn, dn)


def _thresh_body(cm_ref, thr_ref):
    buf = cm_ref[...]
    b = lax.bitcast_convert_type(buf, jnp.int32)
    imin = jnp.int32(-0x80000000)
    key = jnp.where(b >= 0, b, imin - b)
    bound = jnp.int32(0x3F8147AE)
    lo = jnp.full((_QBLK, 1), -bound, jnp.int32)
    hi = jnp.full((_QBLK, 1), bound, jnp.int32)
    for _ in range(31):
        mid = lo + ((hi - lo) >> 1)
        cnt = jnp.sum((key >= mid).astype(jnp.int32), axis=1, keepdims=True)
        ge = cnt >= _K
        lo = jnp.where(ge, mid, lo)
        hi = jnp.where(ge, hi, mid)
    tb = jnp.where(lo >= 0, lo, imin - lo)
    thr_ref[...] = lax.bitcast_convert_type(tb, jnp.float32)


def _thresholds(cm2):
    return pl.pallas_call(
        _thresh_body,
        grid=(_Q // _QBLK,),
        in_specs=[pl.BlockSpec((_QBLK, _NCHUNK), lambda i: (i, 0))],
        out_specs=pl.BlockSpec((_QBLK, 1), lambda i: (i, 0)),
        out_shape=jax.ShapeDtypeStruct((_Q, 1), jnp.float32),
    )(cm2)


_IDCAP = 80


def _sc_gather_filter(s_tab, cm2, thr, cv_out, ci_out,
                      cm_v, ids_v, thr_v, rows_v, out_v, out_i,
                      gsem0, gsem1, csem0, csem1):
    cid = lax.axis_index("c")
    sid = lax.axis_index("s")
    wid = sid * 2 + cid
    qbase = wid * _NQ_PER

    pltpu.sync_copy(thr.at[pl.ds(qbase, _NQ_PER)], thr_v.at[pl.ds(0, _NQ_PER)])

    gsems = (gsem0, gsem1)
    csems = (csem0, csem1)

    iota16 = lax.iota(jnp.int32, 16)
    negv = jnp.full((16,), _NEG, jnp.float32)
    zerov = jnp.zeros((16,), jnp.int32)

    def _cm_start(k, b):
        pltpu.async_copy(cm2.at[pl.ds(qbase + k * _BATCH, _BATCH)],
                         cm_v.at[b], csems[b])

    def _cm_wait(k, b):
        pltpu.make_async_copy(cm2.at[pl.ds(qbase + k * _BATCH, _BATCH)],
                              cm_v.at[b], csems[b]).wait()

    def _gather_start(k, b):
        for u in range(_BATCH):
            pltpu.async_copy(s_tab.at[ids_v.at[b, u, pl.ds(0, _K)]],
                             rows_v.at[b, pl.ds(u * _K, _K)], gsems[b])

    def _gather_wait(k, b):
        for u in range(_BATCH):
            pltpu.make_async_copy(s_tab.at[ids_v.at[b, u, pl.ds(0, _K)]],
                                  rows_v.at[b, pl.ds(u * _K, _K)],
                                  gsems[b]).wait()

    def _make_ids(k, b):
        def _mq(u, carry):
            t = thr_v[pl.ds(k * _BATCH + u, 16)][0]
            qg = qbase + k * _BATCH + u
            for i in range(_IDCAP // 16):
                ids_v[b, u, pl.ds(i * 16, 16)] = qg + jnp.zeros((16,),
                                                                jnp.int32)

            def _g7(g, cnt):
                for g2 in range(7):
                    gg = g * 7 + g2
                    v = cm_v[b, u, pl.ds(gg * 16, 16)]
                    mask = v >= t
                    tot = plsc.all_reduce_population_count(mask)[0]
                    rowv = (gg * 16 + iota16) * _Q + qg
                    plsc.store_compressed(ids_v.at[b, u, pl.ds(cnt, 16)],
                                          rowv, mask=mask)
                    cnt = jnp.minimum(cnt + tot, _IDCAP - 16)
                return cnt

            lax.fori_loop(0, 7, _g7, jnp.int32(0))
            return carry

        lax.fori_loop(0, _BATCH, _mq, jnp.int32(0))

    _cm_start(0, 0)
    _cm_start(1, 1)
    _cm_wait(0, 0)
    _make_ids(0, 0)
    _gather_start(0, 0)
    _cm_wait(1, 1)
    _make_ids(1, 1)
    _gather_start(1, 1)
    _cm_start(2, 0)
    _cm_start(3, 1)

    def _process(k, b):
        _gather_wait(k, b)

        def _query(u, carry):
            for i in range(_CAP // 16):
                out_v[u, pl.ds(i * 16, 16)] = negv
                out_i[u, pl.ds(i * 16, 16)] = zerov
            t = thr_v[pl.ds(k * _BATCH + u, 16)][0]
            qg = qbase + k * _BATCH + u

            def _group(g, cnt):
                rvec = ids_v[b, u, pl.ds(g * 16, 16)]
                bases = (rvec - qg) >> 5
                for rg in range(16):
                    base = bases[rg]
                    row = u * _K + g * 16 + rg
                    for j in range(8):
                        v = rows_v[b, row, pl.ds(j * 16, 16)]
                        mask = v >= t
                        tot = plsc.all_reduce_population_count(mask)[0]
                        idxv = base + j * 16 + iota16
                        plsc.store_compressed(out_v.at[u, pl.ds(cnt, 16)],
                                              v, mask=mask)
                        plsc.store_compressed(out_i.at[u, pl.ds(cnt, 16)],
                                              idxv, mask=mask)
                        cnt = jnp.minimum(cnt + tot, _CAP - 16)
                return cnt

            lax.fori_loop(0, _K // 16, _group, jnp.int32(0))
            return carry

        lax.fori_loop(0, _BATCH, _query, jnp.int32(0))
        pltpu.sync_copy(out_v, cv_out.at[pl.ds(qbase + k * _BATCH, _BATCH)])
        pltpu.sync_copy(out_i, ci_out.at[pl.ds(qbase + k * _BATCH, _BATCH)])

        @pl.when(k + 2 < _NBATCH)
        def _():
            _cm_wait(k + 2, b)
            _make_ids(k + 2, b)
            _gather_start(k + 2, b)

        @pl.when(k + 4 < _NBATCH)
        def _():
            _cm_start(k + 4, b)

    def _pair(h, carry):
        _process(2 * h, 0)
        _process(2 * h + 1, 1)
        return carry

    lax.fori_loop(0, _NBATCH // 2, _pair, jnp.int32(0))


def _sc_candidates(s_tab, cm2, thr):
    mesh = plsc.VectorSubcoreMesh(core_axis_name="c", subcore_axis_name="s")
    f = functools.partial(
        pl.kernel,
        mesh=mesh,
        compiler_params=pltpu.CompilerParams(
            use_tc_tiling_on_sc=False, needs_layout_passes=False),
        out_type=[
            jax.ShapeDtypeStruct((_Q, _CAP), jnp.float32),
            jax.ShapeDtypeStruct((_Q, _CAP), jnp.int32),
        ],
        scratch_types=[
            pltpu.VMEM((2, _BATCH, _NCHUNK), jnp.float32),
            pltpu.VMEM((2, _BATCH, _IDCAP), jnp.int32),
            pltpu.VMEM((_NQ_PER + 16, ), jnp.float32),
            pltpu.VMEM((2, _BATCH * _K, 128), jnp.float32),
            pltpu.VMEM((_BATCH, _CAP), jnp.float32),
            pltpu.VMEM((_BATCH, _CAP), jnp.int32),
            pltpu.SemaphoreType.DMA,
            pltpu.SemaphoreType.DMA,
            pltpu.SemaphoreType.DMA,
            pltpu.SemaphoreType.DMA,
        ],
    )(_sc_gather_filter)
    return f(s_tab, cm2, thr)


def _final_body(cv_ref, ci_ref, val_ref, idx_ref):
    buf = cv_ref[...]
    ibuf = ci_ref[...]
    lanes = lax.broadcasted_iota(jnp.int32, buf.shape, 1)
    oh = lax.broadcasted_iota(jnp.int32, (_QBLK, _K), 1)
    vacc = jnp.zeros((_QBLK, _K), jnp.float32)
    iacc = jnp.zeros((_QBLK, _K), jnp.int32)
    big = jnp.int32(2**30)
    for it in range(_K):
        m = jnp.max(buf, axis=1, keepdims=True)
        sel = buf == m
        didx = jnp.min(jnp.where(sel, ibuf, big), axis=1, keepdims=True)
        buf = jnp.where(sel & (ibuf == didx), _NEG, buf)
        vacc = vacc + jnp.where(oh == it, m, 0.0)
        iacc = iacc + jnp.where(oh == it, didx, 0)
    val_ref[...] = vacc
    idx_ref[...] = iacc


def _final_topk(cv, ci):
    return pl.pallas_call(
        _final_body,
        grid=(_Q // _QBLK,),
        in_specs=[
            pl.BlockSpec((_QBLK, _CAP), lambda i: (i, 0)),
            pl.BlockSpec((_QBLK, _CAP), lambda i: (i, 0)),
        ],
        out_specs=[
            pl.BlockSpec((_QBLK, _K), lambda i: (i, 0)),
            pl.BlockSpec((_QBLK, _K), lambda i: (i, 0)),
        ],
        out_shape=[
            jax.ShapeDtypeStruct((_Q, _K), jnp.float32),
            jax.ShapeDtypeStruct((_Q, _K), jnp.int32),
        ],
    )(cv, ci)


def kernel(query_embed, doc_embeds, k):
    n = doc_embeds.shape[0]

    def _l2n(x):
        nrm = jnp.linalg.norm(x, ord=2, axis=-1, keepdims=True)
        return x / jnp.maximum(nrm, 1e-12)

    qn = _l2n(query_embed)
    dn = jnp.pad(_l2n(doc_embeds), ((0, _DPAD - n), (0, 0)))

    s3, cm = _similarity(qn, dn, n)
    cm2 = jnp.transpose(cm, (1, 0, 2)).reshape(_Q, _NCHUNK)
    thr = _thresholds(cm2)

    s_tab = s3.reshape(_NCHUNK * _Q, 128)
    cv, ci = _sc_candidates(s_tab, cm2, thr.reshape(_Q))

    values, indices = _final_topk(cv, ci)
    return (indices, values)


# speedup vs baseline: 1.5651x; 1.0392x over previous
import functools

import jax
import jax.numpy as jnp
from jax import lax
from jax.experimental import pallas as pl
from jax.experimental.pallas import tpu as pltpu
from jax.experimental.pallas import tpu_sc as plsc

_NEG = -1.0e30

_Q = 4096
_E = 128
_DPAD = 100352
_NCHUNK = 784
_QBLK = 256
_DBLK = 3584
_K = 64
_CAP = 128

_NWORKER = 32
_NQ_PER = _Q // _NWORKER
_BATCH = 4
_NBATCH = _NQ_PER // _BATCH


def _sim_body(n_valid, nj, q_ref, d_ref, s_ref, cm_ref):
    j = pl.program_id(0)
    s = lax.dot_general(
        q_ref[...], d_ref[...],
        dimension_numbers=(((1,), (1,)), ((), ())),
        preferred_element_type=jnp.float32,
    )

    def _write(vals):
        s3 = vals.reshape(_QBLK, _DBLK // 128, 128)
        s_ref[...] = jnp.swapaxes(s3, 0, 1)
        cm_ref[...] = jnp.max(s3, axis=2).reshape(1, _QBLK, _DBLK // 128)

    @pl.when(j < nj - 1)
    def _():
        _write(s)

    @pl.when(j == nj - 1)
    def _():
        doc_idx = j * _DBLK + lax.broadcasted_iota(jnp.int32, s.shape, 1)
        _write(jnp.where(doc_idx < n_valid, s, _NEG))


def _similarity(qn, dn, n_valid):
    nj = _DPAD // _DBLK
    ni = _Q // _QBLK
    body = functools.partial(_sim_body, n_valid, nj)
    return pl.pallas_call(
        body,
        grid=(nj, ni),
        in_specs=[
            pl.BlockSpec((_QBLK, _E), lambda j, i: (i, 0)),
            pl.BlockSpec((_DBLK, _E), lambda j, i: (j, 0)),
        ],
        out_specs=[
            pl.BlockSpec((_DBLK // 128, _QBLK, 128), lambda j, i: (j, i, 0)),
            pl.BlockSpec((1, _QBLK, _DBLK // 128), lambda j, i: (j, i, 0)),
        ],
        out_shape=[
            jax.ShapeDtypeStruct((_NCHUNK, _Q, 128), jnp.float32),
            jax.ShapeDtypeStruct((nj, _Q, _DBLK // 128), jnp.float32),
        ],
    )(qn, dn)


def _thresh_body(cm_ref, thr_ref):
    buf = cm_ref[...]
    b = lax.bitcast_convert_type(buf, jnp.int32)
    imin = jnp.int32(-0x80000000)
    key = jnp.where(b >= 0, b, imin - b)
    bound = jnp.int32(0x3F8147AE)
    lo = jnp.full((_QBLK, 1), -bound, jnp.int32)
    hi = jnp.full((_QBLK, 1), bound, jnp.int32)
    for _ in range(31):
        mid = lo + ((hi - lo) >> 1)
        cnt = jnp.sum((key >= mid).astype(jnp.int32), axis=1, keepdims=True)
        ge = cnt >= _K
        lo = jnp.where(ge, mid, lo)
        hi = jnp.where(ge, hi, mid)
    tb = jnp.where(lo >= 0, lo, imin - lo)
    thr_ref[...] = lax.bitcast_convert_type(tb, jnp.float32)


def _thresholds(cm2):
    return pl.pallas_call(
        _thresh_body,
        grid=(_Q // _QBLK,),
        in_specs=[pl.BlockSpec((_QBLK, _NCHUNK), lambda i: (i, 0))],
        out_specs=pl.BlockSpec((_QBLK, 1), lambda i: (i, 0)),
        out_shape=jax.ShapeDtypeStruct((_Q, 1), jnp.float32),
    )(cm2)


_IDCAP = 80


def _sc_gather_filter(s_tab, cm2, thr, cv_out, ci_out,
                      cm_v, ids_v, thr_v, rows_v, out_v, out_i,
                      gsem0, gsem1, csem0, csem1):
    cid = lax.axis_index("c")
    sid = lax.axis_index("s")
    wid = sid * 2 + cid
    qbase = wid * _NQ_PER

    pltpu.sync_copy(thr.at[pl.ds(qbase, _NQ_PER)], thr_v.at[pl.ds(0, _NQ_PER)])

    gsems = (gsem0, gsem1)
    csems = (csem0, csem1)

    iota16 = lax.iota(jnp.int32, 16)
    negv = jnp.full((16,), _NEG, jnp.float32)
    zerov = jnp.zeros((16,), jnp.int32)

    def _cm_start(k, b):
        pltpu.async_copy(cm2.at[pl.ds(qbase + k * _BATCH, _BATCH)],
                         cm_v.at[b], csems[b])

    def _cm_wait(k, b):
        pltpu.make_async_copy(cm2.at[pl.ds(qbase + k * _BATCH, _BATCH)],
                              cm_v.at[b], csems[b]).wait()

    def _gather_start(k, b):
        for u in range(_BATCH):
            pltpu.async_copy(s_tab.at[ids_v.at[b, u, pl.ds(0, _K)]],
                             rows_v.at[b, pl.ds(u * _K, _K)], gsems[b])

    def _gather_wait(k, b):
        for u in range(_BATCH):
            pltpu.make_async_copy(s_tab.at[ids_v.at[b, u, pl.ds(0, _K)]],
                                  rows_v.at[b, pl.ds(u * _K, _K)],
                                  gsems[b]).wait()

    def _make_ids(k, b):
        def _mq(u, carry):
            t = thr_v[pl.ds(k * _BATCH + u, 16)][0]
            qg = qbase + k * _BATCH + u
            for i in range(_IDCAP // 16):
                ids_v[b, u, pl.ds(i * 16, 16)] = qg + jnp.zeros((16,),
                                                                jnp.int32)

            def _g7(g, cnt):
                for g2 in range(7):
                    gg = g * 7 + g2
                    v = cm_v[b, u, pl.ds(gg * 16, 16)]
                    mask = v >= t
                    tot = plsc.all_reduce_population_count(mask)[0]
                    rowv = (gg * 16 + iota16) * _Q + qg
                    plsc.store_compressed(ids_v.at[b, u, pl.ds(cnt, 16)],
                                          rowv, mask=mask)
                    cnt = jnp.minimum(cnt + tot, _IDCAP - 16)
                return cnt

            lax.fori_loop(0, 7, _g7, jnp.int32(0))
            return carry

        lax.fori_loop(0, _BATCH, _mq, jnp.int32(0))

    _cm_start(0, 0)
    _cm_start(1, 1)
    _cm_wait(0, 0)
    _make_ids(0, 0)
    _gather_start(0, 0)
    _cm_wait(1, 1)
    _make_ids(1, 1)
    _gather_start(1, 1)
    _cm_start(2, 0)
    _cm_start(3, 1)

    def _process(k, b):
        _gather_wait(k, b)

        def _query(u, carry):
            for i in range(_CAP // 16):
                out_v[u, pl.ds(i * 16, 16)] = negv
                out_i[u, pl.ds(i * 16, 16)] = zerov
            t = thr_v[pl.ds(k * _BATCH + u, 16)][0]
            qg = qbase + k * _BATCH + u

            def _group(g, cnt):
                rvec = ids_v[b, u, pl.ds(g * 16, 16)]
                bases = (rvec - qg) >> 5
                for rg in range(16):
                    base = bases[rg]
                    row = u * _K + g * 16 + rg
                    for j in range(8):
                        v = rows_v[b, row, pl.ds(j * 16, 16)]
                        mask = v >= t
                        tot = plsc.all_reduce_population_count(mask)[0]
                        idxv = base + j * 16 + iota16
                        plsc.store_compressed(out_v.at[u, pl.ds(cnt, 16)],
                                              v, mask=mask)
                        plsc.store_compressed(out_i.at[u, pl.ds(cnt, 16)],
                                              idxv, mask=mask)
                        cnt = jnp.minimum(cnt + tot, _CAP - 16)
                return cnt

            lax.fori_loop(0, _K // 16, _group, jnp.int32(0))
            return carry

        lax.fori_loop(0, _BATCH, _query, jnp.int32(0))
        pltpu.sync_copy(out_v, cv_out.at[pl.ds(qbase + k * _BATCH, _BATCH)])
        pltpu.sync_copy(out_i, ci_out.at[pl.ds(qbase + k * _BATCH, _BATCH)])

        @pl.when(k + 2 < _NBATCH)
        def _():
            _cm_wait(k + 2, b)
            _make_ids(k + 2, b)
            _gather_start(k + 2, b)

        @pl.when(k + 4 < _NBATCH)
        def _():
            _cm_start(k + 4, b)

    def _pair(h, carry):
        _process(2 * h, 0)
        _process(2 * h + 1, 1)
        return carry

    lax.fori_loop(0, _NBATCH // 2, _pair, jnp.int32(0))


def _sc_candidates(s_tab, cm2, thr):
    mesh = plsc.VectorSubcoreMesh(core_axis_name="c", subcore_axis_name="s")
    f = functools.partial(
        pl.kernel,
        mesh=mesh,
        compiler_params=pltpu.CompilerParams(
            use_tc_tiling_on_sc=False, needs_layout_passes=False),
        out_type=[
            jax.ShapeDtypeStruct((_Q, _CAP), jnp.float32),
            jax.ShapeDtypeStruct((_Q, _CAP), jnp.int32),
        ],
        scratch_types=[
            pltpu.VMEM((2, _BATCH, _NCHUNK), jnp.float32),
            pltpu.VMEM((2, _BATCH, _IDCAP), jnp.int32),
            pltpu.VMEM((_NQ_PER + 16, ), jnp.float32),
            pltpu.VMEM((2, _BATCH * _K, 128), jnp.float32),
            pltpu.VMEM((_BATCH, _CAP), jnp.float32),
            pltpu.VMEM((_BATCH, _CAP), jnp.int32),
            pltpu.SemaphoreType.DMA,
            pltpu.SemaphoreType.DMA,
            pltpu.SemaphoreType.DMA,
            pltpu.SemaphoreType.DMA,
        ],
    )(_sc_gather_filter)
    return f(s_tab, cm2, thr)


def _final_body(cv_ref, ci_ref, val_ref, idx_ref):
    buf = cv_ref[...]
    ibuf = ci_ref[...]
    lanes = lax.broadcasted_iota(jnp.int32, buf.shape, 1)
    oh = lax.broadcasted_iota(jnp.int32, (_QBLK, _K), 1)
    vacc = jnp.zeros((_QBLK, _K), jnp.float32)
    iacc = jnp.zeros((_QBLK, _K), jnp.int32)
    big = jnp.int32(2**30)
    for it in range(_K):
        m = jnp.max(buf, axis=1, keepdims=True)
        sel = buf == m
        didx = jnp.min(jnp.where(sel, ibuf, big), axis=1, keepdims=True)
        buf = jnp.where(sel & (ibuf == didx), _NEG, buf)
        vacc = vacc + jnp.where(oh == it, m, 0.0)
        iacc = iacc + jnp.where(oh == it, didx, 0)
    val_ref[...] = vacc
    idx_ref[...] = iacc


def _final_topk(cv, ci):
    return pl.pallas_call(
        _final_body,
        grid=(_Q // _QBLK,),
        in_specs=[
            pl.BlockSpec((_QBLK, _CAP), lambda i: (i, 0)),
            pl.BlockSpec((_QBLK, _CAP), lambda i: (i, 0)),
        ],
        out_specs=[
            pl.BlockSpec((_QBLK, _K), lambda i: (i, 0)),
            pl.BlockSpec((_QBLK, _K), lambda i: (i, 0)),
        ],
        out_shape=[
            jax.ShapeDtypeStruct((_Q, _K), jnp.float32),
            jax.ShapeDtypeStruct((_Q, _K), jnp.int32),
        ],
    )(cv, ci)


def kernel(query_embed, doc_embeds, k):
    n = doc_embeds.shape[0]

    def _l2n(x):
        nrm = jnp.linalg.norm(x, ord=2, axis=-1, keepdims=True)
        return x / jnp.maximum(nrm, 1e-12)

    qn = _l2n(query_embed)
    dn = jnp.pad(_l2n(doc_embeds), ((0, _DPAD - n), (0, 0)))

    s3, cm = _similarity(qn, dn, n)
    cm2 = jnp.transpose(cm, (1, 0, 2)).reshape(_Q, _NCHUNK)
    thr = _thresholds(cm2)

    s_tab = s3.reshape(_NCHUNK * _Q, 128)
    cv, ci = _sc_candidates(s_tab, cm2, thr.reshape(_Q))

    values, indices = _final_topk(cv, ci)
    return (indices, values)
